# baseline (speedup 1.0000x reference)
import functools

import jax
import jax.numpy as jnp
from jax import lax
from jax.experimental import pallas as pl
from jax.experimental.pallas import tpu as pltpu
from jax.experimental.pallas import tpu_sc as plsc

NUM_WORKERS = 32
LANES = 16


BLK_V = 384
BLK_R = BLK_V // 2


def _transpose_slab(in_ref, out_ref, rows, cols):
  iota = lax.iota(jnp.int32, LANES)

  def row_body(r, carry):
    for h in range(8):
      a = h // 4
      f0 = (h % 4) * LANES
      col = 2 * r + a

      @pl.when(col < cols)
      def _():
        vals = plsc.load_gather(
            in_ref, [f0 + iota, jnp.full((LANES,), col, jnp.int32)])
        out_ref[r, pl.ds(a * 64 + f0, LANES)] = vals
    return carry

  lax.fori_loop(0, rows, row_body, 0, unroll=False)


def _detile_body(tbl_t, tail_slab, t128, in_v, out_v, tail_in, tail_out,
                 sem_in, sem_out, *, vocab):
  wid = lax.axis_index("s") * 2 + lax.axis_index("c")
  n_blocks_total = vocab // BLK_V
  n_uniform = (n_blocks_total // NUM_WORKERS) * NUM_WORKERS
  n = n_uniform // NUM_WORKERS

  def blk_id(g):
    return g * NUM_WORKERS + wid

  def in_desc(g, b):
    v0 = blk_id(g) * BLK_V
    return pltpu.make_async_copy(tbl_t.at[:, pl.ds(v0, BLK_V)], in_v.at[b],
                                 sem_in.at[b])

  def out_desc(g, b):
    r0 = blk_id(g) * BLK_R
    return pltpu.make_async_copy(out_v.at[b], t128.at[pl.ds(r0, BLK_R)],
                                 sem_out.at[b])

  in_desc(0, 0).start()

  def body(g, carry):
    b = lax.rem(g, 2)

    @pl.when(g >= 2)
    def _():
      out_desc(g - 2, b).wait()

    in_desc(g, b).wait()

    @pl.when(g + 1 < n)
    def _():
      in_desc(g + 1, 1 - b).start()

    _transpose_slab(in_v.at[b], out_v.at[b], BLK_R, BLK_V)
    out_desc(g, b).start()
    return carry

  lax.fori_loop(0, n, body, 0, unroll=False)
  for k in range(2):
    g = n - 2 + k
    out_desc(g, lax.rem(g, 2)).wait()

  blk_extra = n_uniform + wid

  @pl.when(blk_extra < n_blocks_total)
  def _():
    v0 = blk_extra * BLK_V
    pltpu.sync_copy(tbl_t.at[:, pl.ds(v0, BLK_V)], in_v.at[0])
    _transpose_slab(in_v.at[0], out_v.at[0], BLK_R, BLK_V)
    pltpu.sync_copy(out_v.at[0], t128.at[pl.ds(blk_extra * BLK_R, BLK_R)])

  tail = vocab - n_blocks_total * BLK_V
  if tail:
    @pl.when(wid == 1)
    def _():
      v0 = n_blocks_total * BLK_V
      pltpu.sync_copy(tail_slab, tail_in)
      _transpose_slab(tail_in, tail_out, tail // 2, tail)
      pltpu.sync_copy(tail_out, t128.at[pl.ds(v0 // 2, tail // 2)])


SAMPLES_PER_CHUNK = 4
NBUF = 2


def _gather_body(idx_hbm, table_hbm, out_hbm, idx_v, rows_v, sem_idx,
                 sem_gather, sem_out, *, chunks_per_worker, hist):
  wid = lax.axis_index("s") * 2 + lax.axis_index("c")
  samp0 = wid * (chunks_per_worker * SAMPLES_PER_CHUNK)
  n = chunks_per_worker

  def chunk_start(g):
    return pl.multiple_of(samp0 + g * SAMPLES_PER_CHUNK, SAMPLES_PER_CHUNK)

  def idx_desc(g, b):
    return pltpu.make_async_copy(
        idx_hbm.at[pl.ds(chunk_start(g), SAMPLES_PER_CHUNK)], idx_v.at[b],
        sem_idx.at[b])

  def gather_descs(b):
    return [
        pltpu.make_async_copy(
            table_hbm.at[idx_v.at[b, i]],
            rows_v.at[b, i],
            sem_gather.at[b],
        ) for i in range(SAMPLES_PER_CHUNK)
    ]

  def out_desc(g, b):
    return pltpu.make_async_copy(
        rows_v.at[b], out_hbm.at[pl.ds(chunk_start(g), SAMPLES_PER_CHUNK)],
        sem_out.at[b])

  idx_desc(0, 0).start()

  def body(g, carry):
    b = lax.rem(g, NBUF)

    @pl.when(g >= NBUF)
    def _():
      out_desc(g - NBUF, b).wait()

    idx_desc(g, b).wait()
    for d in gather_descs(b):
      d.start()

    @pl.when(g + 1 < n)
    def _():
      idx_desc(g + 1, 1 - b).start()

    for d in gather_descs(b):
      d.wait()
    out_desc(g, b).start()
    return carry

  lax.fori_loop(0, n, body, 0, unroll=False)
  for k in range(NBUF):
    g = n - NBUF + k
    out_desc(g, lax.rem(g, NBUF)).wait()


def kernel(indices, table):
  batch, hist = indices.shape
  vocab, embed_dim = table.shape
  assert embed_dim == 64 and vocab % 2 == 0
  assert batch % (NUM_WORKERS * SAMPLES_PER_CHUNK) == 0
  chunks_per_worker = batch // (NUM_WORKERS * SAMPLES_PER_CHUNK)

  mesh = plsc.VectorSubcoreMesh(core_axis_name="c", subcore_axis_name="s")
  compact = pltpu.CompilerParams(use_tc_tiling_on_sc=True,
                                 needs_layout_passes=False)

  detile = functools.partial(
      pl.kernel,
      mesh=mesh,
      out_type=jax.ShapeDtypeStruct((vocab // 2, 128), jnp.float32),
      scratch_types=[
          pltpu.VMEM((2, embed_dim, BLK_V), jnp.float32),
          pltpu.VMEM((2, BLK_R, 128), jnp.float32),
          pltpu.VMEM((embed_dim, 64), jnp.float32),
          pltpu.VMEM((32, 128), jnp.float32),
          pltpu.SemaphoreType.DMA((2,)),
          pltpu.SemaphoreType.DMA((2,)),
      ],
      compiler_params=compact,
  )(functools.partial(_detile_body, vocab=vocab))

  n_tail = vocab - (vocab // BLK_V) * BLK_V
  tail_slab = jnp.swapaxes(
      lax.slice(table, (vocab - n_tail, 0), (vocab, embed_dim)), 0, 1)
  t128 = detile(table.T, tail_slab)

  gather = functools.partial(
      pl.kernel,
      mesh=mesh,
      out_type=jax.ShapeDtypeStruct((batch, hist, embed_dim), jnp.float32),
      scratch_types=[
          pltpu.VMEM((NBUF, SAMPLES_PER_CHUNK, hist), jnp.int32),
          pltpu.VMEM((NBUF, SAMPLES_PER_CHUNK, hist, embed_dim), jnp.float32),
          pltpu.SemaphoreType.DMA((NBUF,)),
          pltpu.SemaphoreType.DMA((NBUF,)),
          pltpu.SemaphoreType.DMA((NBUF,)),
      ],
      compiler_params=pltpu.CompilerParams(use_tc_tiling_on_sc=False),
  )(functools.partial(
      _gather_body, chunks_per_worker=chunks_per_worker, hist=hist))

  return gather(indices.astype(jnp.int32), t128.reshape(vocab, embed_dim))


# speedup vs baseline: 1.1244x; 1.1244x over previous
import functools

import jax
import jax.numpy as jnp
from jax import lax
from jax.experimental import pallas as pl
from jax.experimental.pallas import tpu as pltpu
from jax.experimental.pallas import tpu_sc as plsc

NUM_WORKERS = 32
LANES = 16


BLK_V = 384
BLK_R = BLK_V // 2


BLK_F = BLK_V * 64


def _detile_body(tbl_t, tail_slab, t128, in0, in1, out0, out1, tail_in,
                 sem_in, sem_out, *, vocab):
  wid = lax.axis_index("s") * 2 + lax.axis_index("c")
  n_blocks_total = vocab // BLK_V
  n_uniform = (n_blocks_total // NUM_WORKERS) * NUM_WORKERS
  n = n_uniform // NUM_WORKERS
  ins = (in0, in1)
  outs = (out0, out1)
  iota = lax.iota(jnp.int32, LANES)
  pattern = lax.shift_right_logical(iota, 1) * 128 + (iota & 1) * 64

  def blk_id(g):
    return g * NUM_WORKERS + wid

  def in_descs(s, g):
    v0 = blk_id(g) * BLK_V
    return [
        pltpu.make_async_copy(tbl_t.at[f, pl.ds(v0, BLK_V)],
                              ins[s].at[pl.ds(f * BLK_V, BLK_V)],
                              sem_in.at[s]) for f in range(64)
    ]

  def out_desc(s, g):
    return pltpu.make_async_copy(outs[s],
                                 t128.at[pl.ds(blk_id(g) * BLK_F, BLK_F)],
                                 sem_out.at[s])

  def compute(s):
    def fbody(f, carry):
      for c0 in range(0, BLK_V, LANES):
        vals = ins[s][pl.ds(f * BLK_V + c0, LANES)]
        idx = pattern + ((c0 // 2) * 128 + f)
        plsc.store_scatter(outs[s], [idx], vals)
      return carry

    lax.fori_loop(0, 64, fbody, 0, unroll=False)

  def stage(s, g, fire_next, next_valid):
    for d in in_descs(s, g):
      d.wait()

    @pl.when(next_valid)
    def _():
      for d in in_descs(1 - s, g + 1):
        d.start()

    @pl.when(g >= 2)
    def _():
      out_desc(s, g - 2).wait()

    compute(s)
    out_desc(s, g).start()

  for d in in_descs(0, 0):
    d.start()

  def body(g2, carry):
    ga = 2 * g2
    gb = ga + 1
    stage(0, ga, True, gb < n)

    @pl.when(gb < n)
    def _():
      stage(1, gb, True, gb + 1 < n)

    return carry

  lax.fori_loop(0, (n + 1) // 2, body, 0, unroll=False)
  out_desc(0, n - 1).wait()
  if n > 1:
    out_desc(1, n - 2).wait()

  @pl.when(blk_id(n) < n_blocks_total)
  def _():
    for d in in_descs(0, n):
      d.start()
    for d in in_descs(0, n):
      d.wait()
    compute(0)
    out_desc(0, n).start()
    out_desc(0, n).wait()

  tail = vocab - n_blocks_total * BLK_V
  if tail:
    @pl.when(wid == 1)
    def _():
      pltpu.sync_copy(tail_slab, tail_in)

      def tbody(f, carry):
        for c0 in range(0, tail, LANES):
          vals = tail_in[f, pl.ds(c0, LANES)]
          idx = pattern + ((c0 // 2) * 128 + f)
          plsc.store_scatter(out1, [idx], vals)
        return carry

      lax.fori_loop(0, 64, tbody, 0, unroll=False)
      pltpu.sync_copy(out1.at[pl.ds(0, tail * 64)],
                      t128.at[pl.ds(n_blocks_total * BLK_F, tail * 64)])


SAMPLES_PER_CHUNK = 4
NBUF = 2


def _gather_body(idx_hbm, table_hbm, out_hbm, idx_v, rows_v, sem_idx,
                 sem_gather, sem_out, *, chunks_per_worker, hist):
  wid = lax.axis_index("s") * 2 + lax.axis_index("c")
  samp0 = wid * (chunks_per_worker * SAMPLES_PER_CHUNK)
  n = chunks_per_worker

  def chunk_start(g):
    return pl.multiple_of(samp0 + g * SAMPLES_PER_CHUNK, SAMPLES_PER_CHUNK)

  def idx_desc(g, b):
    return pltpu.make_async_copy(
        idx_hbm.at[pl.ds(chunk_start(g), SAMPLES_PER_CHUNK)], idx_v.at[b],
        sem_idx.at[b])

  def gather_descs(b):
    return [
        pltpu.make_async_copy(
            table_hbm.at[idx_v.at[b, i]],
            rows_v.at[b, i],
            sem_gather.at[b],
        ) for i in range(SAMPLES_PER_CHUNK)
    ]

  def out_desc(g, b):
    return pltpu.make_async_copy(
        rows_v.at[b], out_hbm.at[pl.ds(chunk_start(g), SAMPLES_PER_CHUNK)],
        sem_out.at[b])

  idx_desc(0, 0).start()

  def body(g, carry):
    b = lax.rem(g, NBUF)

    @pl.when(g >= NBUF)
    def _():
      out_desc(g - NBUF, b).wait()

    idx_desc(g, b).wait()
    for d in gather_descs(b):
      d.start()

    @pl.when(g + 1 < n)
    def _():
      idx_desc(g + 1, 1 - b).start()

    for d in gather_descs(b):
      d.wait()
    out_desc(g, b).start()
    return carry

  lax.fori_loop(0, n, body, 0, unroll=False)
  for k in range(NBUF):
    g = n - NBUF + k
    out_desc(g, lax.rem(g, NBUF)).wait()


def kernel(indices, table):
  batch, hist = indices.shape
  vocab, embed_dim = table.shape
  assert embed_dim == 64 and vocab % 2 == 0
  assert batch % (NUM_WORKERS * SAMPLES_PER_CHUNK) == 0
  chunks_per_worker = batch // (NUM_WORKERS * SAMPLES_PER_CHUNK)

  mesh = plsc.VectorSubcoreMesh(core_axis_name="c", subcore_axis_name="s")
  compact = pltpu.CompilerParams(use_tc_tiling_on_sc=True,
                                 needs_layout_passes=False)

  detile = functools.partial(
      pl.kernel,
      mesh=mesh,
      out_type=jax.ShapeDtypeStruct((vocab * embed_dim,), jnp.float32),
      scratch_types=[
          pltpu.VMEM((BLK_F,), jnp.float32),
          pltpu.VMEM((BLK_F,), jnp.float32),
          pltpu.VMEM((BLK_F,), jnp.float32),
          pltpu.VMEM((BLK_F,), jnp.float32),
          pltpu.VMEM((embed_dim, 64), jnp.float32),
          pltpu.SemaphoreType.DMA((2,)),
          pltpu.SemaphoreType.DMA((2,)),
      ],
      compiler_params=compact,
  )(functools.partial(_detile_body, vocab=vocab))

  n_tail = vocab - (vocab // BLK_V) * BLK_V
  tail_slab = jnp.swapaxes(
      lax.slice(table, (vocab - n_tail, 0), (vocab, embed_dim)), 0, 1)
  t128 = detile(table.T, tail_slab)

  gather = functools.partial(
      pl.kernel,
      mesh=mesh,
      out_type=jax.ShapeDtypeStruct((batch, hist, embed_dim), jnp.float32),
      scratch_types=[
          pltpu.VMEM((NBUF, SAMPLES_PER_CHUNK, hist), jnp.int32),
          pltpu.VMEM((NBUF, SAMPLES_PER_CHUNK, hist, embed_dim), jnp.float32),
          pltpu.SemaphoreType.DMA((NBUF,)),
          pltpu.SemaphoreType.DMA((NBUF,)),
          pltpu.SemaphoreType.DMA((NBUF,)),
      ],
      compiler_params=pltpu.CompilerParams(use_tc_tiling_on_sc=False),
  )(functools.partial(
      _gather_body, chunks_per_worker=chunks_per_worker, hist=hist))

  return gather(indices.astype(jnp.int32), t128.reshape(vocab, embed_dim))


# speedup vs baseline: 1.1451x; 1.0184x over previous
import functools

import jax
import jax.numpy as jnp
from jax import lax
from jax.experimental import pallas as pl
from jax.experimental.pallas import tpu as pltpu
from jax.experimental.pallas import tpu_sc as plsc

NUM_WORKERS = 32
LANES = 16


BLK_V = 384
BLK_R = BLK_V // 2


BLK_F = BLK_V * 64


def _detile_body(tbl_t, tail_slab, t128, in0, in1, out0, out1, tail_in,
                 sem_in, sem_out, *, vocab):
  wid = lax.axis_index("s") * 2 + lax.axis_index("c")
  n_blocks_total = vocab // BLK_V
  n_uniform = (n_blocks_total // NUM_WORKERS) * NUM_WORKERS
  n = n_uniform // NUM_WORKERS
  ins = (in0, in1)
  outs = (out0, out1)
  iota = lax.iota(jnp.int32, LANES)
  pattern = lax.shift_right_logical(iota, 1) * 128 + (iota & 1) * 64

  def blk_id(g):
    return g * NUM_WORKERS + wid

  def in_descs(s, g):
    v0 = blk_id(g) * BLK_V
    return [
        pltpu.make_async_copy(tbl_t.at[:, pl.ds(v0, BLK_V)], ins[s],
                              sem_in.at[s])
    ]

  def out_desc(s, g):
    return pltpu.make_async_copy(outs[s],
                                 t128.at[pl.ds(blk_id(g) * BLK_F, BLK_F)],
                                 sem_out.at[s])

  def compute(s):
    def cbody(cc, carry):
      c0 = cc * LANES
      base = pattern + lax.shift_right_logical(c0, 1) * 128
      for f in range(64):
        vals = ins[s][f, pl.ds(c0, LANES)]
        plsc.store_scatter(outs[s], [base + f], vals)
      return carry

    lax.fori_loop(0, BLK_V // LANES, cbody, 0, unroll=False)

  def stage(s, g, fire_next, next_valid):
    for d in in_descs(s, g):
      d.wait()

    @pl.when(next_valid)
    def _():
      for d in in_descs(1 - s, g + 1):
        d.start()

    @pl.when(g >= 2)
    def _():
      out_desc(s, g - 2).wait()

    compute(s)
    out_desc(s, g).start()

  for d in in_descs(0, 0):
    d.start()

  def body(g2, carry):
    ga = 2 * g2
    gb = ga + 1
    stage(0, ga, True, gb < n)

    @pl.when(gb < n)
    def _():
      stage(1, gb, True, gb + 1 < n)

    return carry

  lax.fori_loop(0, (n + 1) // 2, body, 0, unroll=False)
  out_desc(0, n - 1).wait()
  if n > 1:
    out_desc(1, n - 2).wait()

  @pl.when(blk_id(n) < n_blocks_total)
  def _():
    for d in in_descs(0, n):
      d.start()
    for d in in_descs(0, n):
      d.wait()
    compute(0)
    out_desc(0, n).start()
    out_desc(0, n).wait()

  tail = vocab - n_blocks_total * BLK_V
  if tail:
    @pl.when(wid == 1)
    def _():
      pltpu.sync_copy(tail_slab, tail_in)

      for f in range(64):
        for c0 in range(0, tail, LANES):
          vals = tail_in[f, pl.ds(c0, LANES)]
          idx = pattern + ((c0 // 2) * 128 + f)
          plsc.store_scatter(out1, [idx], vals)
      pltpu.sync_copy(out1.at[pl.ds(0, tail * 64)],
                      t128.at[pl.ds(n_blocks_total * BLK_F, tail * 64)])


SAMPLES_PER_CHUNK = 4
NBUF = 2


def _gather_body(idx_hbm, table_hbm, out_hbm, idx_v, rows_v, sem_idx,
                 sem_gather, sem_out, *, chunks_per_worker, hist):
  wid = lax.axis_index("s") * 2 + lax.axis_index("c")
  samp0 = wid * (chunks_per_worker * SAMPLES_PER_CHUNK)
  n = chunks_per_worker

  def chunk_start(g):
    return pl.multiple_of(samp0 + g * SAMPLES_PER_CHUNK, SAMPLES_PER_CHUNK)

  def idx_desc(g, b):
    return pltpu.make_async_copy(
        idx_hbm.at[pl.ds(chunk_start(g), SAMPLES_PER_CHUNK)], idx_v.at[b],
        sem_idx.at[b])

  def gather_descs(b):
    return [
        pltpu.make_async_copy(
            table_hbm.at[idx_v.at[b, i]],
            rows_v.at[b, i],
            sem_gather.at[b],
        ) for i in range(SAMPLES_PER_CHUNK)
    ]

  def out_desc(g, b):
    return pltpu.make_async_copy(
        rows_v.at[b], out_hbm.at[pl.ds(chunk_start(g), SAMPLES_PER_CHUNK)],
        sem_out.at[b])

  idx_desc(0, 0).start()

  def body(g, carry):
    b = lax.rem(g, NBUF)

    @pl.when(g >= NBUF)
    def _():
      out_desc(g - NBUF, b).wait()

    idx_desc(g, b).wait()
    for d in gather_descs(b):
      d.start()

    @pl.when(g + 1 < n)
    def _():
      idx_desc(g + 1, 1 - b).start()

    for d in gather_descs(b):
      d.wait()
    out_desc(g, b).start()
    return carry

  lax.fori_loop(0, n, body, 0, unroll=False)
  for k in range(NBUF):
    g = n - NBUF + k
    out_desc(g, lax.rem(g, NBUF)).wait()


def kernel(indices, table):
  batch, hist = indices.shape
  vocab, embed_dim = table.shape
  assert embed_dim == 64 and vocab % 2 == 0
  assert batch % (NUM_WORKERS * SAMPLES_PER_CHUNK) == 0
  chunks_per_worker = batch // (NUM_WORKERS * SAMPLES_PER_CHUNK)

  mesh = plsc.VectorSubcoreMesh(core_axis_name="c", subcore_axis_name="s")
  compact = pltpu.CompilerParams(use_tc_tiling_on_sc=True,
                                 needs_layout_passes=False)

  detile = functools.partial(
      pl.kernel,
      mesh=mesh,
      out_type=jax.ShapeDtypeStruct((vocab * embed_dim,), jnp.float32),
      scratch_types=[
          pltpu.VMEM((embed_dim, BLK_V), jnp.float32),
          pltpu.VMEM((embed_dim, BLK_V), jnp.float32),
          pltpu.VMEM((BLK_F,), jnp.float32),
          pltpu.VMEM((BLK_F,), jnp.float32),
          pltpu.VMEM((embed_dim, 64), jnp.float32),
          pltpu.SemaphoreType.DMA((2,)),
          pltpu.SemaphoreType.DMA((2,)),
      ],
      compiler_params=compact,
  )(functools.partial(_detile_body, vocab=vocab))

  n_tail = vocab - (vocab // BLK_V) * BLK_V
  tail_slab = jnp.swapaxes(
      lax.slice(table, (vocab - n_tail, 0), (vocab, embed_dim)), 0, 1)
  t128 = detile(table.T, tail_slab)

  gather = functools.partial(
      pl.kernel,
      mesh=mesh,
      out_type=jax.ShapeDtypeStruct((batch, hist, embed_dim), jnp.float32),
      scratch_types=[
          pltpu.VMEM((NBUF, SAMPLES_PER_CHUNK, hist), jnp.int32),
          pltpu.VMEM((NBUF, SAMPLES_PER_CHUNK, hist, embed_dim), jnp.float32),
          pltpu.SemaphoreType.DMA((NBUF,)),
          pltpu.SemaphoreType.DMA((NBUF,)),
          pltpu.SemaphoreType.DMA((NBUF,)),
      ],
      compiler_params=pltpu.CompilerParams(use_tc_tiling_on_sc=False),
  )(functools.partial(
      _gather_body, chunks_per_worker=chunks_per_worker, hist=hist))

  return gather(indices.astype(jnp.int32), t128.reshape(vocab, embed_dim))


# speedup vs baseline: 1.7421x; 1.5213x over previous
import functools

import jax
import jax.numpy as jnp
from jax import lax
from jax.experimental import pallas as pl
from jax.experimental.pallas import tpu as pltpu
from jax.experimental.pallas import tpu_sc as plsc

NUM_WORKERS = 32
SAMPLES_PER_CHUNK = 8
NBUF = 2


def _gather_body(idx_hbm, table_hbm, out_hbm, idx_v, rows_v, sem_idx,
                 sem_gather, sem_out, *, chunks_per_worker, hist):
  wid = lax.axis_index("s") * 2 + lax.axis_index("c")
  samp0 = wid * (chunks_per_worker * SAMPLES_PER_CHUNK)
  n = chunks_per_worker

  def chunk_start(g):
    return pl.multiple_of(samp0 + g * SAMPLES_PER_CHUNK, SAMPLES_PER_CHUNK)

  def start_idx_load(g, b):
    pltpu.async_copy(idx_hbm.at[pl.ds(chunk_start(g), SAMPLES_PER_CHUNK)],
                     idx_v.at[b], sem_idx.at[b])

  def wait_idx_load(g, b):
    pltpu.make_async_copy(
        idx_hbm.at[pl.ds(chunk_start(g), SAMPLES_PER_CHUNK)], idx_v.at[b],
        sem_idx.at[b]).wait()

  def gather_descs(b):
    return [
        pltpu.make_async_copy(
            table_hbm.at[idx_v.at[b, i]],
            rows_v.at[b, i],
            sem_gather.at[b],
        ) for i in range(SAMPLES_PER_CHUNK)
    ]

  def out_desc(g, b):
    return pltpu.make_async_copy(
        rows_v.at[b],
        out_hbm.at[pl.ds(chunk_start(g), SAMPLES_PER_CHUNK)], sem_out.at[b])

  start_idx_load(0, 0)

  def body(g, carry):
    b = lax.rem(g, NBUF)
    @pl.when(g >= NBUF)
    def _():
      out_desc(g - NBUF, b).wait()

    wait_idx_load(g, b)
    for d in gather_descs(b):
      d.start()

    @pl.when(g + 1 < n)
    def _():
      start_idx_load(g + 1, 1 - b)

    for d in gather_descs(b):
      d.wait()
    out_desc(g, b).start()
    return carry

  lax.fori_loop(0, n, body, 0, unroll=False)

  for k in range(NBUF):
    g = n - NBUF + k
    out_desc(g, lax.rem(g, NBUF)).wait()


def kernel(indices, table):
  batch, hist = indices.shape
  _, embed_dim = table.shape
  assert batch % (NUM_WORKERS * SAMPLES_PER_CHUNK) == 0
  chunks_per_worker = batch // (NUM_WORKERS * SAMPLES_PER_CHUNK)

  mesh = plsc.VectorSubcoreMesh(core_axis_name="c", subcore_axis_name="s")
  gather = functools.partial(
      pl.kernel,
      mesh=mesh,
      out_type=jax.ShapeDtypeStruct((batch, hist, embed_dim), jnp.float32),
      scratch_types=[
          pltpu.VMEM((NBUF, SAMPLES_PER_CHUNK, hist), jnp.int32),
          pltpu.VMEM((NBUF, SAMPLES_PER_CHUNK, hist, embed_dim), jnp.float32),
          pltpu.SemaphoreType.DMA((NBUF,)),
          pltpu.SemaphoreType.DMA((NBUF,)),
          pltpu.SemaphoreType.DMA((NBUF,)),
      ],
      compiler_params=pltpu.CompilerParams(use_tc_tiling_on_sc=False),
  )(functools.partial(
      _gather_body, chunks_per_worker=chunks_per_worker, hist=hist))

  return gather(indices.astype(jnp.int32), table)
